# baseline (speedup 1.0000x reference)
import dataclasses
import functools

import jax
import jax.numpy as jnp
from jax import lax
from jax.experimental import pallas as pl
from jax.experimental.pallas import tpu as pltpu
from jax.experimental.pallas import tpu_sc as plsc

NC = 2
NS = 16
NW = NC * NS
LANES = 16

_mesh = functools.partial(
    plsc.VectorSubcoreMesh, core_axis_name="c", subcore_axis_name="s"
)


def _sc_params():
    cp = pltpu.CompilerParams()
    if "needs_layout_passes" in pltpu.CompilerParams.__dataclass_fields__:
        cp = dataclasses.replace(cp, needs_layout_passes=False)
    return cp


def _wid():
    return lax.axis_index("c") * NS + lax.axis_index("s")


def _zero_ref(ref, n):
    z = jnp.zeros((LANES,), jnp.float32)

    @pl.loop(0, n, step=LANES)
    def _(i):
        ref[pl.ds(i, LANES)] = z


def _sc_degree(src, dst, w, n):
    e = src.shape[0]
    epw = e // NW

    @functools.partial(
        pl.kernel,
        out_type=jax.ShapeDtypeStruct((NW, n), jnp.float32),
        mesh=_mesh(),
        compiler_params=_sc_params(),
        scratch_types=[
            pltpu.VMEM((epw,), jnp.int32),
            pltpu.VMEM((epw,), jnp.int32),
            pltpu.VMEM((epw,), jnp.float32),
            pltpu.VMEM((n,), jnp.float32),
        ],
    )
    def k(src_hbm, dst_hbm, w_hbm, out_hbm, sb, db, wb, deg):
        wid = _wid()
        base = wid * epw
        pltpu.sync_copy(src_hbm.at[pl.ds(base, epw)], sb)
        pltpu.sync_copy(dst_hbm.at[pl.ds(base, epw)], db)
        pltpu.sync_copy(w_hbm.at[pl.ds(base, epw)], wb)
        _zero_ref(deg, n)

        @pl.loop(0, epw, step=LANES)
        def _(i):
            s = sb[pl.ds(i, LANES)]
            d = db[pl.ds(i, LANES)]
            wv = wb[pl.ds(i, LANES)]
            wz = jnp.where(s == d, 0.0, wv)
            plsc.addupdate_scatter(deg, [s], wz)

        pltpu.sync_copy(deg, out_hbm.at[wid])

    return k(src, dst, w)


def _sc_wnorm(src, dst, w, dinv):
    e = src.shape[0]
    n = dinv.shape[0]
    epw = e // NW

    @functools.partial(
        pl.kernel,
        out_type=jax.ShapeDtypeStruct((e,), jnp.float32),
        mesh=_mesh(),
        compiler_params=_sc_params(),
        scratch_types=[
            pltpu.VMEM((epw,), jnp.int32),
            pltpu.VMEM((epw,), jnp.int32),
            pltpu.VMEM((epw,), jnp.float32),
            pltpu.VMEM((epw,), jnp.float32),
            pltpu.VMEM((n,), jnp.float32),
        ],
    )
    def k(src_hbm, dst_hbm, w_hbm, dinv_hbm, out_hbm, sb, db, wb, ob, dv):
        wid = _wid()
        base = wid * epw
        pltpu.sync_copy(src_hbm.at[pl.ds(base, epw)], sb)
        pltpu.sync_copy(dst_hbm.at[pl.ds(base, epw)], db)
        pltpu.sync_copy(w_hbm.at[pl.ds(base, epw)], wb)
        pltpu.sync_copy(dinv_hbm, dv)

        @pl.loop(0, epw, step=LANES)
        def _(i):
            s = sb[pl.ds(i, LANES)]
            d = db[pl.ds(i, LANES)]
            wv = wb[pl.ds(i, LANES)]
            wz = jnp.where(s == d, 0.0, wv)
            g1 = plsc.load_gather(dv, [s])
            g2 = plsc.load_gather(dv, [d])
            ob[pl.ds(i, LANES)] = g1 * wz * g2

        pltpu.sync_copy(ob, out_hbm.at[pl.ds(base, epw)])

    return k(src, dst, w, dinv)


def _sc_spmm2(xT, src, dst, wn, chunk=4000):
    d, n = xT.shape
    e = src.shape[0]
    cpw = d // NW

    out_t = jax.ShapeDtypeStruct((d, n), jnp.float32)
    col = pltpu.VMEM((n,), jnp.float32)

    nch = e // chunk
    assert nch % 2 == 0

    ibuf = pltpu.VMEM((chunk,), jnp.int32)
    fbuf = pltpu.VMEM((chunk,), jnp.float32)

    @functools.partial(
        pl.kernel,
        out_type=[out_t, out_t],
        mesh=_mesh(),
        compiler_params=_sc_params(),
        scratch_types=[
            [col] * cpw,
            [col] * cpw,
            [[ibuf, ibuf, fbuf] for _ in range(2)],
            [pltpu.SemaphoreType.DMA for _ in range(2)],
        ],
    )
    def k(xT_hbm, src_hbm, dst_hbm, wn_hbm, s1_hbm, s2_hbm, xc, ac, ebuf, sems):
        wid = _wid()
        row = wid * cpw
        for c in range(cpw):
            pltpu.sync_copy(xT_hbm.at[row + c], xc[c])
            _zero_ref(ac[c], n)

        def start(slot, ch):
            for hbm, buf in zip((src_hbm, dst_hbm, wn_hbm), ebuf[slot]):
                pltpu.async_copy(hbm.at[pl.ds(ch, chunk)], buf, sems[slot])

        def wait(slot):
            for hbm, buf in zip((src_hbm, dst_hbm, wn_hbm), ebuf[slot]):
                pltpu.make_async_copy(hbm.at[pl.ds(0, chunk)], buf,
                                      sems[slot]).wait()

        def compute(slot, gsrc, gdst):
            sb, db, wb = ebuf[slot]

            @plsc.parallel_loop(0, chunk, step=LANES, unroll=1)
            def _(i):
                s = sb[pl.ds(i, LANES)]
                dt = db[pl.ds(i, LANES)]
                wv = wb[pl.ds(i, LANES)]
                for c in range(cpw):
                    g = plsc.load_gather(gsrc[c], [s])
                    plsc.addupdate_scatter(gdst[c], [dt], wv * g)

        last = (nch - 1) * chunk

        def spmm(gsrc, gdst):
            start(0, 0)

            @pl.loop(0, nch, step=2)
            def _(ci):
                b = ci * chunk
                start(1, b + chunk)
                wait(0)
                compute(0, gsrc, gdst)
                start(0, jnp.minimum(b + 2 * chunk, last))
                wait(1)
                compute(1, gsrc, gdst)

            wait(0)

        spmm(xc, ac)
        for c in range(cpw):
            pltpu.sync_copy(ac[c], s1_hbm.at[row + c])
            _zero_ref(xc[c], n)
        spmm(ac, xc)
        for c in range(cpw):
            pltpu.sync_copy(xc[c], s2_hbm.at[row + c])

    return k(xT, src, dst, wn)


def _tc_cheb_out(x, s1T, s2T, W, b, blk=2048):
    n, din = x.shape
    dout = W.shape[2]
    hp = jax.lax.Precision.HIGHEST

    def body(x_ref, s1_ref, s2_ref, w_ref, b_ref, o_ref):
        wa = w_ref[0] - w_ref[2]
        acc = jnp.dot(x_ref[...], wa, preferred_element_type=jnp.float32,
                      precision=hp)
        acc -= lax.dot_general(
            s1_ref[...], w_ref[1], (((0,), (0,)), ((), ())),
            preferred_element_type=jnp.float32, precision=hp)
        acc += 2.0 * lax.dot_general(
            s2_ref[...], w_ref[2], (((0,), (0,)), ((), ())),
            preferred_element_type=jnp.float32, precision=hp)
        o_ref[...] = acc + b_ref[...]

    grid = (pl.cdiv(n, blk),)
    return pl.pallas_call(
        body,
        grid=grid,
        in_specs=[
            pl.BlockSpec((blk, din), lambda i: (i, 0)),
            pl.BlockSpec((din, blk), lambda i: (0, i)),
            pl.BlockSpec((din, blk), lambda i: (0, i)),
            pl.BlockSpec(W.shape, lambda i: (0, 0, 0)),
            pl.BlockSpec((1, dout), lambda i: (0, 0)),
        ],
        out_specs=pl.BlockSpec((blk, dout), lambda i: (i, 0)),
        out_shape=jax.ShapeDtypeStruct((n, dout), jnp.float32),
    )(x, s1T, s2T, W, b.reshape(1, dout))


def kernel(x, edge_index, edge_weight, W, b):
    n = x.shape[0]
    src = edge_index[0]
    dst = edge_index[1]

    deg_part = _sc_degree(src, dst, edge_weight, n)
    deg = jnp.sum(deg_part, axis=0)
    dinv = jnp.where(deg > 0, lax.rsqrt(jnp.where(deg > 0, deg, 1.0)), 0.0)

    wn = _sc_wnorm(src, dst, edge_weight, dinv)

    xT = x.T
    s1T, s2T = _sc_spmm2(xT, src, dst, wn)

    return _tc_cheb_out(x, s1T, s2T, W, b)


# speedup vs baseline: 1.1004x; 1.1004x over previous
import dataclasses
import functools

import jax
import jax.numpy as jnp
from jax import lax
from jax.experimental import pallas as pl
from jax.experimental.pallas import tpu as pltpu
from jax.experimental.pallas import tpu_sc as plsc

NC = 2
NS = 16
NW = NC * NS
LANES = 16

_mesh = functools.partial(
    plsc.VectorSubcoreMesh, core_axis_name="c", subcore_axis_name="s"
)


def _sc_params():
    cp = pltpu.CompilerParams()
    if "needs_layout_passes" in pltpu.CompilerParams.__dataclass_fields__:
        cp = dataclasses.replace(cp, needs_layout_passes=False)
    return cp


def _wid():
    return lax.axis_index("c") * NS + lax.axis_index("s")


def _zero_ref(ref, n):
    z = jnp.zeros((LANES,), jnp.float32)

    @plsc.parallel_loop(0, n, step=LANES, unroll=2)
    def _(i):
        ref[pl.ds(i, LANES)] = z


def _sc_degree(src, dst, w, n):
    e = src.shape[0]
    epw = e // NW

    @functools.partial(
        pl.kernel,
        out_type=jax.ShapeDtypeStruct((NW, n), jnp.float32),
        mesh=_mesh(),
        compiler_params=_sc_params(),
        scratch_types=[
            pltpu.VMEM((epw,), jnp.int32),
            pltpu.VMEM((epw,), jnp.int32),
            pltpu.VMEM((epw,), jnp.float32),
            pltpu.VMEM((n,), jnp.float32),
        ],
    )
    def k(src_hbm, dst_hbm, w_hbm, out_hbm, sb, db, wb, deg):
        wid = _wid()
        base = wid * epw
        pltpu.sync_copy(src_hbm.at[pl.ds(base, epw)], sb)
        pltpu.sync_copy(dst_hbm.at[pl.ds(base, epw)], db)
        pltpu.sync_copy(w_hbm.at[pl.ds(base, epw)], wb)
        _zero_ref(deg, n)

        @plsc.parallel_loop(0, epw, step=LANES, unroll=2)
        def _(i):
            s = sb[pl.ds(i, LANES)]
            d = db[pl.ds(i, LANES)]
            wv = wb[pl.ds(i, LANES)]
            wz = jnp.where(s == d, 0.0, wv)
            plsc.addupdate_scatter(deg, [s], wz)

        pltpu.sync_copy(deg, out_hbm.at[wid])

    return k(src, dst, w)


def _sc_wnorm(src, dst, w, dinv):
    e = src.shape[0]
    n = dinv.shape[0]
    epw = e // NW

    @functools.partial(
        pl.kernel,
        out_type=jax.ShapeDtypeStruct((e,), jnp.float32),
        mesh=_mesh(),
        compiler_params=_sc_params(),
        scratch_types=[
            pltpu.VMEM((epw,), jnp.int32),
            pltpu.VMEM((epw,), jnp.int32),
            pltpu.VMEM((epw,), jnp.float32),
            pltpu.VMEM((epw,), jnp.float32),
            pltpu.VMEM((n,), jnp.float32),
        ],
    )
    def k(src_hbm, dst_hbm, w_hbm, dinv_hbm, out_hbm, sb, db, wb, ob, dv):
        wid = _wid()
        base = wid * epw
        pltpu.sync_copy(src_hbm.at[pl.ds(base, epw)], sb)
        pltpu.sync_copy(dst_hbm.at[pl.ds(base, epw)], db)
        pltpu.sync_copy(w_hbm.at[pl.ds(base, epw)], wb)
        pltpu.sync_copy(dinv_hbm, dv)

        @plsc.parallel_loop(0, epw, step=LANES, unroll=2)
        def _(i):
            s = sb[pl.ds(i, LANES)]
            d = db[pl.ds(i, LANES)]
            wv = wb[pl.ds(i, LANES)]
            wz = jnp.where(s == d, 0.0, wv)
            g1 = plsc.load_gather(dv, [s])
            g2 = plsc.load_gather(dv, [d])
            ob[pl.ds(i, LANES)] = g1 * wz * g2

        pltpu.sync_copy(ob, out_hbm.at[pl.ds(base, epw)])

    return k(src, dst, w, dinv)


def _sc_spmm2(xT, src, dst, wn, chunk=4000):
    d, n = xT.shape
    e = src.shape[0]
    cpw = d // NW

    out_t = jax.ShapeDtypeStruct((d, n), jnp.float32)
    col = pltpu.VMEM((n,), jnp.float32)

    nch = e // chunk
    assert nch % 2 == 0

    ibuf = pltpu.VMEM((chunk,), jnp.int32)
    fbuf = pltpu.VMEM((chunk,), jnp.float32)

    @functools.partial(
        pl.kernel,
        out_type=[out_t, out_t],
        mesh=_mesh(),
        compiler_params=_sc_params(),
        scratch_types=[
            [col] * cpw,
            [col] * cpw,
            [[ibuf, ibuf, fbuf] for _ in range(2)],
            [pltpu.SemaphoreType.DMA for _ in range(2)],
        ],
    )
    def k(xT_hbm, src_hbm, dst_hbm, wn_hbm, s1_hbm, s2_hbm, xc, ac, ebuf, sems):
        wid = _wid()
        row = wid * cpw
        for c in range(cpw):
            pltpu.sync_copy(xT_hbm.at[row + c], xc[c])
            _zero_ref(ac[c], n)

        def start(slot, ch):
            for hbm, buf in zip((src_hbm, dst_hbm, wn_hbm), ebuf[slot]):
                pltpu.async_copy(hbm.at[pl.ds(ch, chunk)], buf, sems[slot])

        def wait(slot):
            for hbm, buf in zip((src_hbm, dst_hbm, wn_hbm), ebuf[slot]):
                pltpu.make_async_copy(hbm.at[pl.ds(0, chunk)], buf,
                                      sems[slot]).wait()

        def compute(slot, gsrc, gdst):
            sb, db, wb = ebuf[slot]

            @plsc.parallel_loop(0, chunk, step=LANES, unroll=2)
            def _(i):
                s = sb[pl.ds(i, LANES)]
                dt = db[pl.ds(i, LANES)]
                wv = wb[pl.ds(i, LANES)]
                for c in range(cpw):
                    g = plsc.load_gather(gsrc[c], [s])
                    plsc.addupdate_scatter(gdst[c], [dt], wv * g)

        last = (nch - 1) * chunk

        def spmm(gsrc, gdst):
            start(0, 0)

            @pl.loop(0, nch, step=2)
            def _(ci):
                b = ci * chunk
                start(1, b + chunk)
                wait(0)
                compute(0, gsrc, gdst)
                start(0, jnp.minimum(b + 2 * chunk, last))
                wait(1)
                compute(1, gsrc, gdst)

            wait(0)

        spmm(xc, ac)
        for c in range(cpw):
            pltpu.sync_copy(ac[c], s1_hbm.at[row + c])
            _zero_ref(xc[c], n)
        spmm(ac, xc)
        for c in range(cpw):
            pltpu.sync_copy(xc[c], s2_hbm.at[row + c])

    return k(xT, src, dst, wn)


def _tc_cheb_out(x, s1T, s2T, W, b, blk=2048):
    n, din = x.shape
    dout = W.shape[2]
    hp = jax.lax.Precision.HIGHEST

    def body(x_ref, s1_ref, s2_ref, w_ref, b_ref, o_ref):
        wa = w_ref[0] - w_ref[2]
        acc = jnp.dot(x_ref[...], wa, preferred_element_type=jnp.float32,
                      precision=hp)
        acc -= lax.dot_general(
            s1_ref[...], w_ref[1], (((0,), (0,)), ((), ())),
            preferred_element_type=jnp.float32, precision=hp)
        acc += 2.0 * lax.dot_general(
            s2_ref[...], w_ref[2], (((0,), (0,)), ((), ())),
            preferred_element_type=jnp.float32, precision=hp)
        o_ref[...] = acc + b_ref[...]

    grid = (pl.cdiv(n, blk),)
    return pl.pallas_call(
        body,
        grid=grid,
        in_specs=[
            pl.BlockSpec((blk, din), lambda i: (i, 0)),
            pl.BlockSpec((din, blk), lambda i: (0, i)),
            pl.BlockSpec((din, blk), lambda i: (0, i)),
            pl.BlockSpec(W.shape, lambda i: (0, 0, 0)),
            pl.BlockSpec((1, dout), lambda i: (0, 0)),
        ],
        out_specs=pl.BlockSpec((blk, dout), lambda i: (i, 0)),
        out_shape=jax.ShapeDtypeStruct((n, dout), jnp.float32),
    )(x, s1T, s2T, W, b.reshape(1, dout))


def kernel(x, edge_index, edge_weight, W, b):
    n = x.shape[0]
    src = edge_index[0]
    dst = edge_index[1]

    deg_part = _sc_degree(src, dst, edge_weight, n)
    deg = jnp.sum(deg_part, axis=0)
    dinv = jnp.where(deg > 0, lax.rsqrt(jnp.where(deg > 0, deg, 1.0)), 0.0)

    wn = _sc_wnorm(src, dst, edge_weight, dinv)

    xT = x.T
    s1T, s2T = _sc_spmm2(xT, src, dst, wn)

    return _tc_cheb_out(x, s1T, s2T, W, b)
